# Initial kernel scaffold; baseline (speedup 1.0000x reference)
import jax
import jax.numpy as jnp
from jax.experimental import pallas as pl


def kernel(values, input_points, points0, points1, params):
    raise NotImplementedError("write your pallas kernel here")


# speedup vs baseline: 2.1248x; 2.1248x over previous
import functools

import jax
import jax.numpy as jnp
from jax import lax
from jax.experimental import pallas as pl
from jax.experimental.pallas import tpu as pltpu
from jax.experimental.pallas import tpu_sc as plsc

_K = 16
_NB = 16
_R0 = 0.2
_R1 = 0.2 * (2.0 ** 0.5)
_BLK = 128
_NW = 32
_CH = 512


def _pad_rows(x, n, fill=0.0):
    if x.shape[0] == n:
        return x
    pad = jnp.full((n - x.shape[0],) + x.shape[1:], fill, x.dtype)
    return jnp.concatenate([x, pad], axis=0)


def _round_up(n, m):
    return ((n + m - 1) // m) * m


def _knn_body(po_ref, pix_ref, piy_ref, idx_ref):
    pox = po_ref[:, 0:1]
    poy = po_ref[:, 1:2]
    dx = pix_ref[...] - pox
    dy = piy_ref[...] - poy
    d2 = dx * dx + dy * dy
    coli = lax.broadcasted_iota(jnp.int32, d2.shape, 1)
    big = jnp.int32(2 ** 30)
    for k in range(_K):
        m = jnp.min(d2, axis=1, keepdims=True)
        idxi = jnp.min(jnp.where(d2 == m, coli, big), axis=1, keepdims=True)
        sel = coli == idxi
        d2 = jnp.where(sel, jnp.inf, d2)
        idx_ref[:, k:k + 1] = idxi


def _knn(points_out_pad, points_in):
    n_out_pad = points_out_pad.shape[0]
    n_in = points_in.shape[0]
    n_in_pad = _round_up(n_in, _BLK)
    pix = _pad_rows(points_in[:, 0:1], n_in_pad, 1e30).reshape(1, n_in_pad)
    piy = _pad_rows(points_in[:, 1:2], n_in_pad, 1e30).reshape(1, n_in_pad)
    grid = n_out_pad // _BLK
    return pl.pallas_call(
        _knn_body,
        grid=(grid,),
        in_specs=[
            pl.BlockSpec((_BLK, 3), lambda i: (i, 0)),
            pl.BlockSpec((1, n_in_pad), lambda i: (0, 0)),
            pl.BlockSpec((1, n_in_pad), lambda i: (0, 0)),
        ],
        out_specs=pl.BlockSpec((_BLK, _K), lambda i: (i, 0)),
        out_shape=jax.ShapeDtypeStruct((n_out_pad, _K), jnp.int32),
    )(points_out_pad, pix, piy)


def _gather(table, flat_idx):
    n_flat = flat_idx.shape[0]
    d = table.shape[1]
    b_per_w = n_flat // _NW
    n_iter = b_per_w // _CH
    mesh = plsc.VectorSubcoreMesh(core_axis_name="c", subcore_axis_name="s")

    @functools.partial(
        pl.kernel,
        mesh=mesh,
        out_type=jax.ShapeDtypeStruct((n_flat, d), jnp.float32),
        scratch_types=[
            pltpu.VMEM((_CH,), jnp.int32),
            pltpu.VMEM((_CH, d), jnp.float32),
            pltpu.SemaphoreType.DMA,
        ],
    )
    def gk(table_hbm, idx_hbm, out_hbm, idx_v, rows_v, sem):
        wid = lax.axis_index("s") * mesh.num_cores + lax.axis_index("c")
        base = wid * b_per_w

        @pl.loop(0, n_iter)
        def body(j):
            off = base + j * _CH
            pltpu.sync_copy(idx_hbm.at[pl.ds(off, _CH)], idx_v)
            pltpu.async_copy(table_hbm.at[idx_v], rows_v, sem).wait()
            pltpu.sync_copy(rows_v, out_hbm.at[pl.ds(off, _CH)])

    return gk(table, flat_idx)


def _b16(x):
    return x.astype(jnp.bfloat16).astype(jnp.float32)


def _conv_body(po_ref, g_ref, w1_ref, b1_ref, w2_ref, b2_ref, w_ref, bb_ref,
               out_ref, *, cin, crop, do_relu):
    b = po_ref.shape[0]
    d = g_ref.shape[1]
    g3 = g_ref[...].reshape(b, _K, d)
    pox = po_ref[:, 0:1]
    poy = po_ref[:, 1:2]
    c = po_ref[:, 2:3]
    s = po_ref[:, 3:4]
    nx = g3[:, :, cin]
    ny = g3[:, :, cin + 1]
    cn = g3[:, :, cin + 2]
    sn = g3[:, :, cin + 3]
    dx = nx - pox
    dy = ny - poy
    rx = c * dx + s * dy
    ry = -s * dx + c * dy
    cd = cn * c + sn * s
    sd = sn * c - cn * s
    w1 = w1_ref[...]
    h = (rx[:, :, None] * w1[0].reshape(1, 1, 32)
         + ry[:, :, None] * w1[1].reshape(1, 1, 32)
         + cd[:, :, None] * w1[2].reshape(1, 1, 32)
         + sd[:, :, None] * w1[3].reshape(1, 1, 32)
         + b1_ref[...].reshape(1, 1, 32))
    h = jnp.maximum(h, 0.0)
    bas = jnp.dot(h.reshape(b * _K, 32), w2_ref[...],
                  preferred_element_type=jnp.float32) + b2_ref[...]
    dist = jnp.sqrt(rx * rx + ry * ry)
    mask = (dist <= crop).astype(jnp.float32)
    bas3 = bas.reshape(b, _K, _NB) * mask[:, :, None]
    v3 = g3[:, :, :cin]
    prod = bas3[:, :, :, None] * v3[:, :, None, :]
    tmp = jnp.sum(prod, axis=1)
    out = jnp.dot(tmp.reshape(b, _NB * cin), w_ref[...],
                  preferred_element_type=jnp.float32) / _K + bb_ref[...]
    if do_relu:
        out = jnp.maximum(out, 0.0)
    out_ref[...] = out


def _conv(p, points_in, vals_in, points_out_pad, idx_pad, crop, do_relu, n_out):
    n_in = points_in.shape[0]
    n_out_pad = points_out_pad.shape[0]
    cin = vals_in.shape[1]
    cout = p['b'].shape[0]
    d = 128
    th_in = points_in[:, 2]
    table = jnp.concatenate(
        [vals_in, points_in[:, :2],
         jnp.cos(th_in)[:, None], jnp.sin(th_in)[:, None],
         jnp.zeros((n_in, d - cin - 4), jnp.float32)], axis=1)
    g = _gather(table, idx_pad.reshape(-1))
    po_ext = jnp.concatenate(
        [points_out_pad[:, :2],
         jnp.cos(points_out_pad[:, 2:3]), jnp.sin(points_out_pad[:, 2:3])],
        axis=1)
    grid = n_out_pad // _BLK
    body = functools.partial(_conv_body, cin=cin, crop=crop, do_relu=do_relu)
    out = pl.pallas_call(
        body,
        grid=(grid,),
        in_specs=[
            pl.BlockSpec((_BLK, 4), lambda i: (i, 0)),
            pl.BlockSpec((_BLK * _K, d), lambda i: (i, 0)),
            pl.BlockSpec((4, 32), lambda i: (0, 0)),
            pl.BlockSpec((1, 32), lambda i: (0, 0)),
            pl.BlockSpec((32, _NB), lambda i: (0, 0)),
            pl.BlockSpec((1, _NB), lambda i: (0, 0)),
            pl.BlockSpec((_NB * cin, cout), lambda i: (0, 0)),
            pl.BlockSpec((1, cout), lambda i: (0, 0)),
        ],
        out_specs=pl.BlockSpec((_BLK, cout), lambda i: (i, 0)),
        out_shape=jax.ShapeDtypeStruct((n_out_pad, cout), jnp.float32),
    )(po_ext, g, p['W1'], p['b1'].reshape(1, 32), p['W2'],
      p['b2'].reshape(1, _NB), p['W'].reshape(_NB * cin, cout),
      p['b'].reshape(1, cout))
    return out[:n_out]


def _bn(p, x, eps=1e-5):
    m = jnp.mean(x, axis=0)
    v = jnp.var(x, axis=0)
    return (x - m) / jnp.sqrt(v + eps) * p['gamma'] + p['beta']


def kernel(values, input_points, points0, points1, params):
    n0 = points0.shape[0]
    n1 = points1.shape[0]
    pad_q = max(_BLK, (_NW * _CH) // _K)
    n0p = _round_up(n0, pad_q)
    n1p = _round_up(n1, pad_q)
    p0p = _pad_rows(points0, n0p)
    p1p = _pad_rows(points1, n1p)

    radii = (_R0, _R1)
    idx_first = _knn(p0p, input_points)
    v = _conv(params['first'], input_points, values, p0p, idx_first,
              radii[0], True, n0)

    pts = (points0, points1)
    pts_pad = (p0p, p1p)
    n_cur = (n0, n1)
    idx_blk = None
    for i in range(2):
        vin = _bn(params['bn'][i][0], v)
        idx_blk = _knn(pts_pad[i], pts[i])
        v = _conv(params['blocks'][i][0], pts[i], vin, pts_pad[i], idx_blk,
                  radii[i], True, n_cur[i])
        for j in range(1, 3):
            v = _bn(params['bn'][i][j], v)
            v = _conv(params['blocks'][i][j], pts[i], v, pts_pad[i], idx_blk,
                      radii[i], True, n_cur[i])
        v = v + vin
        if i == 0:
            v = _bn(params['bn'][i][3], v)
            idx_down = _knn(p1p, points0)
            v = _conv(params['down'][0], points0, v, p1p, idx_down,
                      radii[1], True, n1)

    v = _bn(params['last_bn'], v)
    v = _conv(params['last'], points1, v, p1p, idx_blk, radii[1], False, n1)
    return jnp.mean(v, axis=0)
